# Initial kernel scaffold; baseline (speedup 1.0000x reference)
import jax
import jax.numpy as jnp
from jax.experimental import pallas as pl


def kernel(state_id, city_id, state_table, city_table, W, b, gamma, beta):
    raise NotImplementedError("write your pallas kernel here")


# speedup vs baseline: 1.2799x; 1.2799x over previous
import functools

import jax
import jax.numpy as jnp
from jax import lax
from jax.experimental import pallas as pl
from jax.experimental.pallas import tpu as pltpu
from jax.experimental.pallas import tpu_sc as plsc

_BATCH = 16384
_STATE_DIM = 32
_CITY_DIM = 64
_CH = 128


def _sc_gather(state_id2d, city_id2d, state_table, city_table):
    info = plsc.get_sparse_core_info()
    nw = info.num_cores * info.num_subcores
    n_rows = _BATCH // _CH
    rows_per_w = n_rows // nw
    mesh = plsc.VectorSubcoreMesh(core_axis_name="c", subcore_axis_name="s")

    @functools.partial(
        pl.kernel,
        mesh=mesh,
        compiler_params=pltpu.CompilerParams(use_tc_tiling_on_sc=False),
        out_type=[
            jax.ShapeDtypeStruct((n_rows, _CH, _STATE_DIM), jnp.float32),
            jax.ShapeDtypeStruct((n_rows, _CH, _CITY_DIM), jnp.float32),
        ],
        scratch_types=[
            pltpu.VMEM((rows_per_w, _CH), jnp.int32),
            pltpu.VMEM((rows_per_w, _CH), jnp.int32),
            pltpu.VMEM((rows_per_w, _CH, _STATE_DIM), jnp.float32),
            pltpu.VMEM((rows_per_w, _CH, _CITY_DIM), jnp.float32),
            pltpu.SemaphoreType.DMA,
        ],
    )
    def gather_k(sid_hbm, cid_hbm, stab_hbm, ctab_hbm, s_out, c_out,
                 idx_s, idx_c, rows_s, rows_c, sem):
        wid = lax.axis_index("s") * info.num_cores + lax.axis_index("c")
        base = wid * rows_per_w
        pltpu.sync_copy(sid_hbm.at[pl.ds(base, rows_per_w)], idx_s)
        pltpu.sync_copy(cid_hbm.at[pl.ds(base, rows_per_w)], idx_c)
        copies = []
        for j in range(rows_per_w):
            copies.append(
                pltpu.async_copy(stab_hbm.at[idx_s.at[j]], rows_s.at[j], sem))
            copies.append(
                pltpu.async_copy(ctab_hbm.at[idx_c.at[j]], rows_c.at[j], sem))
        for cp in copies:
            cp.wait()
        pltpu.sync_copy(rows_s, s_out.at[pl.ds(base, rows_per_w)])
        pltpu.sync_copy(rows_c, c_out.at[pl.ds(base, rows_per_w)])

    return gather_k(state_id2d, city_id2d, state_table, city_table)


def _tc_body(s_ref, c_ref, ws_ref, wc_ref, bgb_ref, o_ref):
    f = (jnp.dot(s_ref[...], ws_ref[...], preferred_element_type=jnp.float32)
         + jnp.dot(c_ref[...], wc_ref[...], preferred_element_type=jnp.float32)
         + bgb_ref[0:1, :])
    mean = jnp.mean(f, axis=-1, keepdims=True)
    var = jnp.mean((f - mean) * (f - mean), axis=-1, keepdims=True)
    o_ref[...] = ((f - mean) * lax.rsqrt(var + 1e-5) * bgb_ref[1:2, :]
                  + bgb_ref[2:3, :])


def _tc_fuse(s_emb, c_emb, W, b, gamma, beta):
    blk = 2048
    bgb = jnp.stack([b, gamma, beta])
    ws = W[:_STATE_DIM]
    wc = W[_STATE_DIM:]
    return pl.pallas_call(
        _tc_body,
        grid=(_BATCH // blk,),
        in_specs=[
            pl.BlockSpec((blk, _STATE_DIM), lambda i: (i, 0)),
            pl.BlockSpec((blk, _CITY_DIM), lambda i: (i, 0)),
            pl.BlockSpec((_STATE_DIM, _CITY_DIM), lambda i: (0, 0)),
            pl.BlockSpec((_CITY_DIM, _CITY_DIM), lambda i: (0, 0)),
            pl.BlockSpec((3, _CITY_DIM), lambda i: (0, 0)),
        ],
        out_specs=pl.BlockSpec((blk, _CITY_DIM), lambda i: (i, 0)),
        out_shape=jax.ShapeDtypeStruct((_BATCH, _CITY_DIM), jnp.float32),
    )(s_emb, c_emb, ws, wc, bgb)


def kernel(state_id, city_id, state_table, city_table, W, b, gamma, beta):
    sid2d = state_id.astype(jnp.int32).reshape(_BATCH // _CH, _CH)
    cid2d = city_id.astype(jnp.int32).reshape(_BATCH // _CH, _CH)
    s_emb, c_emb = _sc_gather(sid2d, cid2d, state_table, city_table)
    s_emb = s_emb.reshape(_BATCH, _STATE_DIM)
    c_emb = c_emb.reshape(_BATCH, _CITY_DIM)
    return _tc_fuse(s_emb, c_emb, W, b, gamma, beta)


# speedup vs baseline: 2.4934x; 1.9482x over previous
import functools

import jax
import jax.numpy as jnp
from jax import lax
from jax.experimental import pallas as pl
from jax.experimental.pallas import tpu as pltpu
from jax.experimental.pallas import tpu_sc as plsc

_BATCH = 16384
_STATE_DIM = 32
_CITY_DIM = 64
_NUM_STATES = 1000
_NUM_CITIES = 100000
_CB = 8192


def _sc_gather_t(state_id, city_id, stab_t, ctab_t):
    info = plsc.get_sparse_core_info()
    mesh = plsc.VectorSubcoreMesh(core_axis_name="c", subcore_axis_name="s")
    n_chunks = _BATCH // _CB

    @functools.partial(
        pl.kernel,
        mesh=mesh,
        compiler_params=pltpu.CompilerParams(needs_layout_passes=False),
        out_type=[
            jax.ShapeDtypeStruct((_STATE_DIM, _BATCH), jnp.float32),
            jax.ShapeDtypeStruct((_CITY_DIM, _BATCH), jnp.float32),
        ],
        scratch_types=[
            pltpu.VMEM((_NUM_CITIES,), jnp.float32),
            pltpu.VMEM((_NUM_STATES,), jnp.float32),
            pltpu.VMEM((_BATCH,), jnp.int32),
            pltpu.VMEM((_CB,), jnp.float32),
            pltpu.SemaphoreType.DMA,
        ],
    )
    def gather_k(sid_hbm, cid_hbm, stabt_hbm, ctabt_hbm, s_out, c_out,
                 row_v, srow_v, idx_v, out_v, sem):
        wid = lax.axis_index("s") * info.num_cores + lax.axis_index("c")

        def gather_chunk(table_ref, k):
            @plsc.parallel_loop(0, _CB, step=16, unroll=8)
            def _(i):
                iv = idx_v[pl.ds(k * _CB + i, 16)]
                out_v[pl.ds(i, 16)] = plsc.load_gather(table_ref, [iv])

        cp0 = pltpu.async_copy(ctabt_hbm.at[2 * wid], row_v, sem)
        pltpu.sync_copy(stabt_hbm.at[wid], srow_v)
        pltpu.sync_copy(sid_hbm, idx_v)
        for k in range(n_chunks):
            gather_chunk(srow_v, k)
            pltpu.sync_copy(out_v, s_out.at[wid, pl.ds(k * _CB, _CB)])
        pltpu.sync_copy(cid_hbm, idx_v)
        cp0.wait()
        for r in range(2):
            d = 2 * wid + r
            for k in range(n_chunks):
                gather_chunk(row_v, k)
                pltpu.sync_copy(out_v, c_out.at[d, pl.ds(k * _CB, _CB)])
            if r == 0:
                pltpu.sync_copy(ctabt_hbm.at[2 * wid + 1], row_v)

    return gather_k(state_id, city_id, stab_t, ctab_t)


def _tc_body(s_ref, c_ref, ws_ref, wc_ref, bgb_ref, o_ref):
    f = (lax.dot_general(s_ref[...], ws_ref[...], (((0,), (0,)), ((), ())),
                         preferred_element_type=jnp.float32)
         + lax.dot_general(c_ref[...], wc_ref[...], (((0,), (0,)), ((), ())),
                           preferred_element_type=jnp.float32)
         + bgb_ref[0:1, :])
    mean = jnp.mean(f, axis=-1, keepdims=True)
    var = jnp.mean((f - mean) * (f - mean), axis=-1, keepdims=True)
    o_ref[...] = ((f - mean) * lax.rsqrt(var + 1e-5) * bgb_ref[1:2, :]
                  + bgb_ref[2:3, :])


def _tc_fuse(s_emb_t, c_emb_t, W, b, gamma, beta):
    blk = 2048
    bgb = jnp.stack([b, gamma, beta])
    ws = W[:_STATE_DIM]
    wc = W[_STATE_DIM:]
    return pl.pallas_call(
        _tc_body,
        grid=(_BATCH // blk,),
        in_specs=[
            pl.BlockSpec((_STATE_DIM, blk), lambda i: (0, i)),
            pl.BlockSpec((_CITY_DIM, blk), lambda i: (0, i)),
            pl.BlockSpec((_STATE_DIM, _CITY_DIM), lambda i: (0, 0)),
            pl.BlockSpec((_CITY_DIM, _CITY_DIM), lambda i: (0, 0)),
            pl.BlockSpec((3, _CITY_DIM), lambda i: (0, 0)),
        ],
        out_specs=pl.BlockSpec((blk, _CITY_DIM), lambda i: (i, 0)),
        out_shape=jax.ShapeDtypeStruct((_BATCH, _CITY_DIM), jnp.float32),
    )(s_emb_t, c_emb_t, ws, wc, bgb)


def kernel(state_id, city_id, state_table, city_table, W, b, gamma, beta):
    sid = state_id.astype(jnp.int32)
    cid = city_id.astype(jnp.int32)
    s_emb_t, c_emb_t = _sc_gather_t(sid, cid, state_table.T, city_table.T)
    return _tc_fuse(s_emb_t, c_emb_t, W, b, gamma, beta)


# speedup vs baseline: 2.4970x; 1.0014x over previous
import functools

import jax
import jax.numpy as jnp
from jax import lax
from jax.experimental import pallas as pl
from jax.experimental.pallas import tpu as pltpu
from jax.experimental.pallas import tpu_sc as plsc

_BATCH = 16384
_STATE_DIM = 32
_CITY_DIM = 64
_NUM_STATES = 1000
_NUM_CITIES = 100000
_CB = 8192


def _sc_gather_t(state_id, city_id, stab_t, ctab_t):
    info = plsc.get_sparse_core_info()
    mesh = plsc.VectorSubcoreMesh(core_axis_name="c", subcore_axis_name="s")
    n_chunks = _BATCH // _CB

    @functools.partial(
        pl.kernel,
        mesh=mesh,
        compiler_params=pltpu.CompilerParams(needs_layout_passes=False),
        out_type=[
            jax.ShapeDtypeStruct((_STATE_DIM, _BATCH), jnp.float32),
            jax.ShapeDtypeStruct((_CITY_DIM, _BATCH), jnp.float32),
        ],
        scratch_types=[
            pltpu.VMEM((_NUM_CITIES,), jnp.float32),
            pltpu.VMEM((_NUM_STATES,), jnp.float32),
            pltpu.VMEM((_BATCH,), jnp.int32),
            pltpu.VMEM((_CB,), jnp.float32),
            pltpu.SemaphoreType.DMA,
        ],
    )
    def gather_k(sid_hbm, cid_hbm, stabt_hbm, ctabt_hbm, s_out, c_out,
                 row_v, srow_v, idx_v, out_v, sem):
        wid = lax.axis_index("s") * info.num_cores + lax.axis_index("c")

        def gather_chunk(table_ref, k):
            @plsc.parallel_loop(0, _CB, step=16, unroll=8)
            def _(i):
                iv = idx_v[pl.ds(k * _CB + i, 16)]
                out_v[pl.ds(i, 16)] = plsc.load_gather(table_ref, [iv])

        cp0 = pltpu.async_copy(ctabt_hbm.at[2 * wid], row_v, sem)
        pltpu.sync_copy(stabt_hbm.at[wid], srow_v)
        pltpu.sync_copy(sid_hbm, idx_v)
        for k in range(n_chunks):
            gather_chunk(srow_v, k)
            pltpu.sync_copy(out_v, s_out.at[wid, pl.ds(k * _CB, _CB)])
        pltpu.sync_copy(cid_hbm, idx_v)
        cp0.wait()
        for r in range(2):
            d = 2 * wid + r
            for k in range(n_chunks):
                gather_chunk(row_v, k)
                pltpu.sync_copy(out_v, c_out.at[d, pl.ds(k * _CB, _CB)])
            if r == 0:
                pltpu.sync_copy(ctabt_hbm.at[2 * wid + 1], row_v)

    return gather_k(state_id, city_id, stab_t, ctab_t)


def _tc_body(s_ref, c_ref, wt_ref, bgb_ref, o_ref):
    ws_t = wt_ref[:, :_STATE_DIM]
    wc_t = wt_ref[:, _STATE_DIM:]
    f = (lax.dot_general(s_ref[...], ws_t, (((0,), (1,)), ((), ())),
                         preferred_element_type=jnp.float32)
         + lax.dot_general(c_ref[...], wc_t, (((0,), (1,)), ((), ())),
                           preferred_element_type=jnp.float32)
         + bgb_ref[0:1, :])
    mean = jnp.mean(f, axis=-1, keepdims=True)
    var = jnp.mean((f - mean) * (f - mean), axis=-1, keepdims=True)
    o_ref[...] = ((f - mean) * lax.rsqrt(var + 1e-5) * bgb_ref[1:2, :]
                  + bgb_ref[2:3, :])


def _tc_fuse(s_emb_t, c_emb_t, W, b, gamma, beta):
    blk = 4096
    bgb = jnp.stack([b, gamma, beta])
    w_t = W.T
    return pl.pallas_call(
        _tc_body,
        grid=(_BATCH // blk,),
        in_specs=[
            pl.BlockSpec((_STATE_DIM, blk), lambda i: (0, i)),
            pl.BlockSpec((_CITY_DIM, blk), lambda i: (0, i)),
            pl.BlockSpec((_CITY_DIM, _STATE_DIM + _CITY_DIM), lambda i: (0, 0)),
            pl.BlockSpec((3, _CITY_DIM), lambda i: (0, 0)),
        ],
        out_specs=pl.BlockSpec((blk, _CITY_DIM), lambda i: (i, 0)),
        out_shape=jax.ShapeDtypeStruct((_BATCH, _CITY_DIM), jnp.float32),
    )(s_emb_t, c_emb_t, w_t, bgb)


def kernel(state_id, city_id, state_table, city_table, W, b, gamma, beta):
    sid = state_id.astype(jnp.int32)
    cid = city_id.astype(jnp.int32)
    s_emb_t, c_emb_t = _sc_gather_t(sid, cid, state_table.T, city_table.T)
    return _tc_fuse(s_emb_t, c_emb_t, W, b, gamma, beta)


# speedup vs baseline: 3.1973x; 1.2805x over previous
import functools

import jax
import jax.numpy as jnp
from jax import lax
from jax.experimental import pallas as pl
from jax.experimental.pallas import tpu as pltpu
from jax.experimental.pallas import tpu_sc as plsc

_BATCH = 16384
_STATE_DIM = 32
_CITY_DIM = 64
_NUM_STATES = 1000
_NUM_CITIES = 100000
_CB = 8192


def _sc_gather_t(state_id, city_id, stab_t, ctab_t):
    info = plsc.get_sparse_core_info()
    mesh = plsc.VectorSubcoreMesh(core_axis_name="c", subcore_axis_name="s")
    n_chunks = _BATCH // _CB

    @functools.partial(
        pl.kernel,
        mesh=mesh,
        compiler_params=pltpu.CompilerParams(needs_layout_passes=False),
        out_type=[
            jax.ShapeDtypeStruct((_STATE_DIM, _BATCH), jnp.float32),
            jax.ShapeDtypeStruct((_CITY_DIM, _BATCH), jnp.float32),
        ],
        scratch_types=[
            pltpu.VMEM((_NUM_CITIES,), jnp.float32),
            pltpu.VMEM((_NUM_STATES,), jnp.float32),
            pltpu.VMEM((_BATCH,), jnp.int32),
            pltpu.VMEM((_CB,), jnp.float32),
            pltpu.SemaphoreType.DMA,
        ],
    )
    def gather_k(sid_hbm, cid_hbm, stabt_hbm, ctabt_hbm, s_out, c_out,
                 row_v, srow_v, idx_v, out_v, sem):
        wid = lax.axis_index("s") * info.num_cores + lax.axis_index("c")

        def gather_chunk(table_ref, k):
            @plsc.parallel_loop(0, _CB, step=16, unroll=8)
            def _(i):
                iv = idx_v[pl.ds(k * _CB + i, 16)]
                out_v[pl.ds(i, 16)] = plsc.load_gather(table_ref, [iv])

        cp0 = pltpu.async_copy(ctabt_hbm.at[2 * wid], row_v, sem)
        pltpu.sync_copy(stabt_hbm.at[wid], srow_v)
        pltpu.sync_copy(sid_hbm, idx_v)
        for k in range(n_chunks):
            gather_chunk(srow_v, k)
            pltpu.sync_copy(out_v, s_out.at[wid, pl.ds(k * _CB, _CB)])
        pltpu.sync_copy(cid_hbm, idx_v)
        cp0.wait()
        for r in range(2):
            d = 2 * wid + r
            for k in range(n_chunks):
                gather_chunk(row_v, k)
                pltpu.sync_copy(out_v, c_out.at[d, pl.ds(k * _CB, _CB)])
            if r == 0:
                pltpu.sync_copy(ctabt_hbm.at[2 * wid + 1], row_v)

    return gather_k(state_id, city_id, stab_t, ctab_t)


def _tc_body(s_ref, c_ref, wt_ref, bgb_ref, o_ref):
    ws_t = wt_ref[:, :_STATE_DIM]
    wc_t = wt_ref[:, _STATE_DIM:]
    f = (jnp.dot(ws_t, s_ref[...], preferred_element_type=jnp.float32)
         + jnp.dot(wc_t, c_ref[...], preferred_element_type=jnp.float32)
         + bgb_ref[:, 0:1])
    mean = jnp.mean(f, axis=0, keepdims=True)
    var = jnp.mean((f - mean) * (f - mean), axis=0, keepdims=True)
    o_ref[...] = ((f - mean) * lax.rsqrt(var + 1e-5) * bgb_ref[:, 1:2]
                  + bgb_ref[:, 2:3])


def _tc_fuse(s_emb_t, c_emb_t, W, b, gamma, beta):
    blk = 4096
    bgb = jnp.stack([b, gamma, beta], axis=1)
    w_t = W.T
    out_t = pl.pallas_call(
        _tc_body,
        grid=(_BATCH // blk,),
        in_specs=[
            pl.BlockSpec((_STATE_DIM, blk), lambda i: (0, i)),
            pl.BlockSpec((_CITY_DIM, blk), lambda i: (0, i)),
            pl.BlockSpec((_CITY_DIM, _STATE_DIM + _CITY_DIM), lambda i: (0, 0)),
            pl.BlockSpec((_CITY_DIM, 3), lambda i: (0, 0)),
        ],
        out_specs=pl.BlockSpec((_CITY_DIM, blk), lambda i: (0, i)),
        out_shape=jax.ShapeDtypeStruct((_CITY_DIM, _BATCH), jnp.float32),
    )(s_emb_t, c_emb_t, w_t, bgb)
    return out_t.T


def kernel(state_id, city_id, state_table, city_table, W, b, gamma, beta):
    sid = state_id.astype(jnp.int32)
    cid = city_id.astype(jnp.int32)
    s_emb_t, c_emb_t = _sc_gather_t(sid, cid, state_table.T, city_table.T)
    return _tc_fuse(s_emb_t, c_emb_t, W, b, gamma, beta)


# speedup vs baseline: 3.4912x; 1.0919x over previous
import functools

import jax
import jax.numpy as jnp
from jax import lax
from jax.experimental import pallas as pl
from jax.experimental.pallas import tpu as pltpu
from jax.experimental.pallas import tpu_sc as plsc

_BATCH = 16384
_STATE_DIM = 32
_CITY_DIM = 64
_NUM_STATES = 1000
_NUM_CITIES = 100000
_CB = 8192


def _sc_gather_t(state_id, city_id, stab_t, ctab_t):
    info = plsc.get_sparse_core_info()
    mesh = plsc.VectorSubcoreMesh(core_axis_name="c", subcore_axis_name="s")
    n_chunks = _BATCH // _CB

    @functools.partial(
        pl.kernel,
        mesh=mesh,
        compiler_params=pltpu.CompilerParams(needs_layout_passes=False),
        out_type=[
            jax.ShapeDtypeStruct((_STATE_DIM, _BATCH), jnp.float32),
            jax.ShapeDtypeStruct((_CITY_DIM, _BATCH), jnp.float32),
        ],
        scratch_types=[
            pltpu.VMEM((_NUM_CITIES,), jnp.float32),
            pltpu.VMEM((_NUM_STATES,), jnp.float32),
            pltpu.VMEM((_BATCH,), jnp.int32),
            pltpu.VMEM((_CB,), jnp.float32),
            pltpu.VMEM_SHARED((2, _BATCH), jnp.int32),
            pltpu.SemaphoreType.DMA,
        ],
    )
    def gather_k(sid_hbm, cid_hbm, stabt_hbm, ctabt_hbm, s_out, c_out,
                 row_v, srow_v, idx_v, out_v, ids_sh, sem):
        sub = lax.axis_index("s")
        wid = sub * info.num_cores + lax.axis_index("c")
        @pl.when(sub == 0)
        def _():
            pltpu.sync_copy(sid_hbm, ids_sh.at[0])
            pltpu.sync_copy(cid_hbm, ids_sh.at[1])
        plsc.subcore_barrier()

        def gather_chunk(table_ref, k):
            @plsc.parallel_loop(0, _CB, step=16, unroll=8)
            def _(i):
                iv = idx_v[pl.ds(k * _CB + i, 16)]
                out_v[pl.ds(i, 16)] = plsc.load_gather(table_ref, [iv])

        cp0 = pltpu.async_copy(ctabt_hbm.at[2 * wid], row_v, sem)
        pltpu.sync_copy(stabt_hbm.at[wid], srow_v)
        pltpu.sync_copy(ids_sh.at[0], idx_v)
        for k in range(n_chunks):
            gather_chunk(srow_v, k)
            pltpu.sync_copy(out_v, s_out.at[wid, pl.ds(k * _CB, _CB)])
        pltpu.sync_copy(ids_sh.at[1], idx_v)
        cp0.wait()
        for r in range(2):
            d = 2 * wid + r
            for k in range(n_chunks):
                gather_chunk(row_v, k)
                pltpu.sync_copy(out_v, c_out.at[d, pl.ds(k * _CB, _CB)])
            if r == 0:
                pltpu.sync_copy(ctabt_hbm.at[2 * wid + 1], row_v)

    return gather_k(state_id, city_id, stab_t, ctab_t)


def _tc_body(s_ref, c_ref, wt_ref, bgb_ref, o_ref):
    ws_t = wt_ref[:, :_STATE_DIM]
    wc_t = wt_ref[:, _STATE_DIM:]
    f = (jnp.dot(ws_t, s_ref[...], preferred_element_type=jnp.float32)
         + jnp.dot(wc_t, c_ref[...], preferred_element_type=jnp.float32)
         + bgb_ref[:, 0:1])
    mean = jnp.mean(f, axis=0, keepdims=True)
    var = jnp.mean((f - mean) * (f - mean), axis=0, keepdims=True)
    o_ref[...] = ((f - mean) * lax.rsqrt(var + 1e-5) * bgb_ref[:, 1:2]
                  + bgb_ref[:, 2:3])


def _tc_fuse(s_emb_t, c_emb_t, W, b, gamma, beta):
    blk = 4096
    bgb = jnp.stack([b, gamma, beta], axis=1)
    w_t = W.T
    out_t = pl.pallas_call(
        _tc_body,
        grid=(_BATCH // blk,),
        in_specs=[
            pl.BlockSpec((_STATE_DIM, blk), lambda i: (0, i)),
            pl.BlockSpec((_CITY_DIM, blk), lambda i: (0, i)),
            pl.BlockSpec((_CITY_DIM, _STATE_DIM + _CITY_DIM), lambda i: (0, 0)),
            pl.BlockSpec((_CITY_DIM, 3), lambda i: (0, 0)),
        ],
        out_specs=pl.BlockSpec((_CITY_DIM, blk), lambda i: (0, i)),
        out_shape=jax.ShapeDtypeStruct((_CITY_DIM, _BATCH), jnp.float32),
    )(s_emb_t, c_emb_t, w_t, bgb)
    return out_t.T


def kernel(state_id, city_id, state_table, city_table, W, b, gamma, beta):
    sid = state_id.astype(jnp.int32)
    cid = city_id.astype(jnp.int32)
    s_emb_t, c_emb_t = _sc_gather_t(sid, cid, state_table.T, city_table.T)
    return _tc_fuse(s_emb_t, c_emb_t, W, b, gamma, beta)


# speedup vs baseline: 3.5011x; 1.0028x over previous
import functools

import jax
import jax.numpy as jnp
from jax import lax
from jax.experimental import pallas as pl
from jax.experimental.pallas import tpu as pltpu
from jax.experimental.pallas import tpu_sc as plsc

_BATCH = 16384
_STATE_DIM = 32
_CITY_DIM = 64
_NUM_STATES = 1000
_NUM_CITIES = 100000
_CB = 8192


def _sc_gather_t(state_id, city_id, stab_t, ctab_t):
    info = plsc.get_sparse_core_info()
    mesh = plsc.VectorSubcoreMesh(core_axis_name="c", subcore_axis_name="s")
    n_chunks = _BATCH // _CB

    @functools.partial(
        pl.kernel,
        mesh=mesh,
        compiler_params=pltpu.CompilerParams(needs_layout_passes=False),
        out_type=[
            jax.ShapeDtypeStruct((_STATE_DIM, _BATCH), jnp.float32),
            jax.ShapeDtypeStruct((_CITY_DIM, _BATCH), jnp.float32),
        ],
        scratch_types=[
            pltpu.VMEM((_NUM_CITIES,), jnp.float32),
            pltpu.VMEM((_NUM_STATES,), jnp.float32),
            pltpu.VMEM((_BATCH,), jnp.int32),
            pltpu.VMEM((_CB,), jnp.float32),
            pltpu.VMEM_SHARED((2, _BATCH), jnp.int32),
            pltpu.SemaphoreType.DMA,
        ],
    )
    def gather_k(sid_hbm, cid_hbm, stabt_hbm, ctabt_hbm, s_out, c_out,
                 row_v, srow_v, idx_v, out_v, ids_sh, sem):
        sub = lax.axis_index("s")
        wid = sub * info.num_cores + lax.axis_index("c")
        @pl.when(sub == 0)
        def _():
            pltpu.sync_copy(sid_hbm, ids_sh.at[0])
            pltpu.sync_copy(cid_hbm, ids_sh.at[1])
        plsc.subcore_barrier()

        def gather_chunk(table_ref, k):
            @plsc.parallel_loop(0, _CB, step=16, unroll=8)
            def _(i):
                iv = idx_v[pl.ds(k * _CB + i, 16)]
                out_v[pl.ds(i, 16)] = plsc.load_gather(table_ref, [iv])

        cp0 = pltpu.async_copy(ctabt_hbm.at[2 * wid], row_v, sem)
        pltpu.sync_copy(stabt_hbm.at[wid], srow_v)
        pltpu.sync_copy(ids_sh.at[0], idx_v)

        def state_chunk(k, carry):
            gather_chunk(srow_v, k)
            pltpu.sync_copy(out_v, s_out.at[wid, pl.ds(k * _CB, _CB)])
            return carry

        lax.fori_loop(0, n_chunks, state_chunk, 0)
        pltpu.sync_copy(ids_sh.at[1], idx_v)
        cp0.wait()

        def city_row(r, carry):
            @pl.when(r > 0)
            def _():
                pltpu.sync_copy(ctabt_hbm.at[2 * wid + r], row_v)

            def city_chunk(k, c2):
                gather_chunk(row_v, k)
                pltpu.sync_copy(out_v, c_out.at[2 * wid + r, pl.ds(k * _CB, _CB)])
                return c2

            lax.fori_loop(0, n_chunks, city_chunk, 0)
            return carry

        lax.fori_loop(0, 2, city_row, 0)

    return gather_k(state_id, city_id, stab_t, ctab_t)


def _tc_body(s_ref, c_ref, wt_ref, bgb_ref, o_ref):
    ws_t = wt_ref[:, :_STATE_DIM]
    wc_t = wt_ref[:, _STATE_DIM:]
    f = (jnp.dot(ws_t, s_ref[...], preferred_element_type=jnp.float32)
         + jnp.dot(wc_t, c_ref[...], preferred_element_type=jnp.float32)
         + bgb_ref[:, 0:1])
    mean = jnp.mean(f, axis=0, keepdims=True)
    var = jnp.mean((f - mean) * (f - mean), axis=0, keepdims=True)
    o_ref[...] = ((f - mean) * lax.rsqrt(var + 1e-5) * bgb_ref[:, 1:2]
                  + bgb_ref[:, 2:3])


def _tc_fuse(s_emb_t, c_emb_t, W, b, gamma, beta):
    blk = 4096
    bgb = jnp.stack([b, gamma, beta], axis=1)
    w_t = W.T
    out_t = pl.pallas_call(
        _tc_body,
        grid=(_BATCH // blk,),
        in_specs=[
            pl.BlockSpec((_STATE_DIM, blk), lambda i: (0, i)),
            pl.BlockSpec((_CITY_DIM, blk), lambda i: (0, i)),
            pl.BlockSpec((_CITY_DIM, _STATE_DIM + _CITY_DIM), lambda i: (0, 0)),
            pl.BlockSpec((_CITY_DIM, 3), lambda i: (0, 0)),
        ],
        out_specs=pl.BlockSpec((_CITY_DIM, blk), lambda i: (0, i)),
        out_shape=jax.ShapeDtypeStruct((_CITY_DIM, _BATCH), jnp.float32),
    )(s_emb_t, c_emb_t, w_t, bgb)
    return out_t.T


def kernel(state_id, city_id, state_table, city_table, W, b, gamma, beta):
    sid = state_id.astype(jnp.int32)
    cid = city_id.astype(jnp.int32)
    s_emb_t, c_emb_t = _sc_gather_t(sid, cid, state_table.T, city_table.T)
    return _tc_fuse(s_emb_t, c_emb_t, W, b, gamma, beta)
